# baseline (speedup 1.0000x reference)
import functools

import jax
import jax.numpy as jnp
from jax import lax
from jax.experimental import pallas as pl
from jax.experimental.pallas import tpu as pltpu
from jax.experimental.pallas import tpu_sc as plsc

N = 10000
P = 320000
D = 128
L = 4

NC = 2
NS = 16
NW = NC * NS
EPW = P // NW
E = 80
NCHUNK = EPW // E
NPAD = 10240
RPT = NPAD // NS

BN = 2000


def _mm_body(h_ref, wlin_ref, blin_ref, wtop_ref, bt_ref, h3_ref, g_ref):
    hb = h_ref[...]
    h3_ref[...] = jnp.dot(hb, wlin_ref[...], preferred_element_type=jnp.float32) + blin_ref[...]
    g_ref[...] = -(jnp.dot(hb, wtop_ref[...], preferred_element_type=jnp.float32) + bt_ref[...])


_matmuls = pl.pallas_call(
    _mm_body,
    grid=(N // BN,),
    in_specs=[
        pl.BlockSpec((BN, D), lambda i: (i, 0)),
        pl.BlockSpec((D, D), lambda i: (0, 0)),
        pl.BlockSpec((1, D), lambda i: (0, 0)),
        pl.BlockSpec((D, D), lambda i: (0, 0)),
        pl.BlockSpec((1, D), lambda i: (0, 0)),
    ],
    out_specs=[
        pl.BlockSpec((BN, D), lambda i: (i, 0)),
        pl.BlockSpec((BN, D), lambda i: (i, 0)),
    ],
    out_shape=[
        jax.ShapeDtypeStruct((N, D), jnp.float32),
        jax.ShapeDtypeStruct((N, D), jnp.float32),
    ],
)


def _sc_body(gneg_hbm, p0_hbm, p1_hbm, d0_hbm, d1_hbm, si_hbm, wdegneg_hbm,
             zero_hbm, out_hbm,
             idx0_v, idx1_v, sidx_v, deg0_v, deg1_v, rows0_v, rows1_v, prod_v,
             wdeg_v, acc_sh, sem0, sem1):
    cid = lax.axis_index("c")
    sid = lax.axis_index("s")
    wid = cid * NS + sid

    pltpu.sync_copy(wdegneg_hbm, wdeg_v)
    r0 = sid * RPT
    pltpu.sync_copy(zero_hbm.at[pl.ds(r0, RPT)], acc_sh.at[pl.ds(r0, RPT)])
    plsc.subcore_barrier()

    def chunk(k, carry):
        base = wid * EPW + k * E
        pltpu.sync_copy(p0_hbm.at[pl.ds(base, E)], idx0_v)
        pltpu.sync_copy(p1_hbm.at[pl.ds(base, E)], idx1_v)
        pltpu.sync_copy(si_hbm.at[pl.ds(base, E)], sidx_v)
        pltpu.sync_copy(d0_hbm.at[pl.ds(base * L, E * L)], deg0_v)
        pltpu.sync_copy(d1_hbm.at[pl.ds(base * L, E * L)], deg1_v)
        cp0 = pltpu.async_copy(gneg_hbm.at[idx0_v], rows0_v, sem0)
        cp1 = pltpu.async_copy(gneg_hbm.at[idx1_v], rows1_v, sem1)
        cp0.wait()
        cp1.wait()

        wv = [[wdeg_v[l, pl.ds(c * 16, 16)] for l in range(L)]
              for c in range(D // 16)]

        @plsc.parallel_loop(0, E, unroll=4)
        def edge(e):
            eb = e * L
            d0 = [plsc.load_gather(deg0_v, [jnp.full((16,), eb + l, jnp.int32)])
                  for l in range(L)]
            d1 = [plsc.load_gather(deg1_v, [jnp.full((16,), eb + l, jnp.int32)])
                  for l in range(L)]
            for c in range(D // 16):
                sl = pl.ds(c * 16, 16)
                an0 = rows0_v[e, sl]
                an1 = rows1_v[e, sl]
                for l in range(L):
                    an0 = an0 + d0[l] * wv[c][l]
                    an1 = an1 + d1[l] * wv[c][l]
                prod_v[e, sl] = 1.0 / ((1.0 + jnp.exp(an0)) * (1.0 + jnp.exp(an1)))

        pltpu.sync_copy(prod_v, acc_sh.at[sidx_v], add=True)
        return carry

    lax.fori_loop(0, NCHUNK, chunk, 0)

    plsc.subcore_barrier()
    pltpu.sync_copy(acc_sh.at[pl.ds(r0, RPT)],
                    out_hbm.at[cid, pl.ds(r0, RPT)])


_sc_gate_scatter = functools.partial(
    pl.kernel,
    out_type=jax.ShapeDtypeStruct((NC, NPAD, D), jnp.float32),
    mesh=plsc.VectorSubcoreMesh(core_axis_name="c", subcore_axis_name="s",
                                num_cores=NC, num_subcores=NS),
    compiler_params=pltpu.CompilerParams(needs_layout_passes=False),
    scratch_types=[
        pltpu.VMEM((E,), jnp.int32),
        pltpu.VMEM((E,), jnp.int32),
        pltpu.VMEM((E,), jnp.int32),
        pltpu.VMEM((E * L,), jnp.float32),
        pltpu.VMEM((E * L,), jnp.float32),
        pltpu.VMEM((E, D), jnp.float32),
        pltpu.VMEM((E, D), jnp.float32),
        pltpu.VMEM((E, D), jnp.float32),
        pltpu.VMEM((L, D), jnp.float32),
        pltpu.VMEM_SHARED((NPAD, D), jnp.float32),
        pltpu.SemaphoreType.DMA,
        pltpu.SemaphoreType.DMA,
    ],
)(_sc_body)


def _combine_body(scale_ref, h3_ref, parts_ref, out_ref):
    s = scale_ref[0, 0]
    out_ref[...] = h3_ref[...] + s * (parts_ref[0] + parts_ref[1])


_combine = pl.pallas_call(
    _combine_body,
    grid=(N // BN,),
    in_specs=[
        pl.BlockSpec(memory_space=pltpu.MemorySpace.SMEM),
        pl.BlockSpec((BN, D), lambda i: (i, 0)),
        pl.BlockSpec((NC, BN, D), lambda i: (0, i, 0)),
    ],
    out_specs=pl.BlockSpec((BN, D), lambda i: (i, 0)),
    out_shape=jax.ShapeDtypeStruct((N, D), jnp.float32),
)


def kernel(h, pairs_0, pairs_1, degrees_0, degrees_1, scatter_idx,
           W_lin, b_lin, W_t, b_t, eps):
    p0 = pairs_0.astype(jnp.int32)
    p1 = pairs_1.astype(jnp.int32)
    si = scatter_idx.astype(jnp.int32)
    w_top = W_t[:D]
    w_deg = W_t[D:]
    h3, g = _matmuls(h, W_lin, b_lin.reshape(1, D), w_top, b_t.reshape(1, D))
    parts = _sc_gate_scatter(g, p0, p1, degrees_0.reshape(P * L),
                             degrees_1.reshape(P * L), si, -w_deg,
                             jnp.zeros((NPAD, D), jnp.float32))
    scale = (1.0 + eps).reshape(1, 1)
    return _combine(scale, h3, parts)


# speedup vs baseline: 2.1358x; 2.1358x over previous
import functools

import jax
import jax.numpy as jnp
from jax import lax
from jax.experimental import pallas as pl
from jax.experimental.pallas import tpu as pltpu
from jax.experimental.pallas import tpu_sc as plsc

N = 10000
P = 320000
D = 128
L = 4

NC = 2
NS = 16
NW = NC * NS
EPW = P // NW
E = 80
NCHUNK = EPW // E
NPAD = 10240
RPT = NPAD // NS

BN = 2000


def _mm_body(h_ref, wlin_ref, blin_ref, wtop_ref, bt_ref, h3_ref, g_ref):
    hb = h_ref[...]
    h3_ref[...] = jnp.dot(hb, wlin_ref[...], preferred_element_type=jnp.float32) + blin_ref[...]
    g_ref[...] = -(jnp.dot(hb, wtop_ref[...], preferred_element_type=jnp.float32) + bt_ref[...])


_matmuls = pl.pallas_call(
    _mm_body,
    grid=(N // BN,),
    in_specs=[
        pl.BlockSpec((BN, D), lambda i: (i, 0)),
        pl.BlockSpec((D, D), lambda i: (0, 0)),
        pl.BlockSpec((1, D), lambda i: (0, 0)),
        pl.BlockSpec((D, D), lambda i: (0, 0)),
        pl.BlockSpec((1, D), lambda i: (0, 0)),
    ],
    out_specs=[
        pl.BlockSpec((BN, D), lambda i: (i, 0)),
        pl.BlockSpec((BN, D), lambda i: (i, 0)),
    ],
    out_shape=[
        jax.ShapeDtypeStruct((N, D), jnp.float32),
        jax.ShapeDtypeStruct((N, D), jnp.float32),
    ],
)


NSETS = 2
NITER = 62
assert NSETS * NITER + 1 == NCHUNK


def _sc_body(gneg_hbm, p0_hbm, p1_hbm, d0_hbm, d1_hbm, si_hbm, wdegneg_hbm,
             zero_hbm, out_hbm,
             idx0_s, idx1_s, sidx_s, deg0_s, deg1_s, rows0_s, rows1_s,
             wdeg_v, acc_sh, seml_s, semg_s):
    cid = lax.axis_index("c")
    sid = lax.axis_index("s")
    wid = cid * NS + sid

    pltpu.sync_copy(wdegneg_hbm, wdeg_v)
    r0 = sid * RPT
    pltpu.sync_copy(zero_hbm.at[pl.ds(r0, RPT)], acc_sh.at[pl.ds(r0, RPT)])
    plsc.subcore_barrier()

    def lin_descs(k, j, make):
        base = wid * EPW + k * E
        f = pltpu.make_async_copy if make else pltpu.async_copy
        return [
            f(p0_hbm.at[pl.ds(base, E)], idx0_s[j], seml_s[j]),
            f(p1_hbm.at[pl.ds(base, E)], idx1_s[j], seml_s[j]),
            f(si_hbm.at[pl.ds(base, E)], sidx_s[j], seml_s[j]),
            f(d0_hbm.at[pl.ds(base * L, E * L)], deg0_s[j], seml_s[j]),
            f(d1_hbm.at[pl.ds(base * L, E * L)], deg1_s[j], seml_s[j]),
        ]

    def fire_lin(k, j):
        lin_descs(k, j, False)

    def wait_lin(k, j):
        for d in lin_descs(k, j, True):
            d.wait()

    def gather_descs(j, make):
        f = pltpu.make_async_copy if make else pltpu.async_copy
        return [
            f(gneg_hbm.at[idx0_s[j]], rows0_s[j], semg_s[j]),
            f(gneg_hbm.at[idx1_s[j]], rows1_s[j], semg_s[j]),
        ]

    def fire_gather(j):
        gather_descs(j, False)

    def wait_gather(j):
        for d in gather_descs(j, True):
            d.wait()

    def compute_scatter(j):
        rows0_v, rows1_v = rows0_s[j], rows1_s[j]
        deg0_v, deg1_v = deg0_s[j], deg1_s[j]

        @plsc.parallel_loop(0, E, unroll=2)
        def edge(e):
            eb = e * L
            d0 = [plsc.load_gather(deg0_v, [jnp.full((16,), eb + l, jnp.int32)])
                  for l in range(L)]
            d1 = [plsc.load_gather(deg1_v, [jnp.full((16,), eb + l, jnp.int32)])
                  for l in range(L)]
            for c in range(D // 16):
                sl = pl.ds(c * 16, 16)
                an0 = rows0_v[e, sl]
                an1 = rows1_v[e, sl]
                for l in range(L):
                    w = wdeg_v[l, sl]
                    an0 = an0 + d0[l] * w
                    an1 = an1 + d1[l] * w
                rows0_v[e, sl] = 1.0 / ((1.0 + jnp.exp(an0)) * (1.0 + jnp.exp(an1)))

        pltpu.sync_copy(rows0_v, acc_sh.at[sidx_s[j]], add=True)

    fire_lin(0, 0)
    fire_lin(1, 1)
    wait_lin(0, 0)
    fire_gather(0)

    def body(i, carry):
        k0 = i * NSETS
        for j in range(NSETS):
            k = k0 + j
            jn = (j + 1) % NSETS
            wait_lin(k + 1, jn)
            fire_gather(jn)
            wait_gather(j)
            compute_scatter(j)

            @pl.when(k + NSETS < NCHUNK)
            def _():
                fire_lin(k + NSETS, j)
        return carry

    lax.fori_loop(0, NITER, body, 0)

    wait_gather(0)
    compute_scatter(0)

    plsc.subcore_barrier()
    pltpu.sync_copy(acc_sh.at[pl.ds(r0, RPT)],
                    out_hbm.at[cid, pl.ds(r0, RPT)])


_sc_gate_scatter = functools.partial(
    pl.kernel,
    out_type=jax.ShapeDtypeStruct((NC, NPAD, D), jnp.float32),
    mesh=plsc.VectorSubcoreMesh(core_axis_name="c", subcore_axis_name="s",
                                num_cores=NC, num_subcores=NS),
    compiler_params=pltpu.CompilerParams(needs_layout_passes=False),
    scratch_types=[
        [pltpu.VMEM((E,), jnp.int32)] * NSETS,
        [pltpu.VMEM((E,), jnp.int32)] * NSETS,
        [pltpu.VMEM((E,), jnp.int32)] * NSETS,
        [pltpu.VMEM((E * L,), jnp.float32)] * NSETS,
        [pltpu.VMEM((E * L,), jnp.float32)] * NSETS,
        [pltpu.VMEM((E, D), jnp.float32)] * NSETS,
        [pltpu.VMEM((E, D), jnp.float32)] * NSETS,
        pltpu.VMEM((L, D), jnp.float32),
        pltpu.VMEM_SHARED((NPAD, D), jnp.float32),
        [pltpu.SemaphoreType.DMA] * NSETS,
        [pltpu.SemaphoreType.DMA] * NSETS,
    ],
)(_sc_body)


def _combine_body(scale_ref, h3_ref, parts_ref, out_ref):
    s = scale_ref[0, 0]
    out_ref[...] = h3_ref[...] + s * (parts_ref[0] + parts_ref[1])


_combine = pl.pallas_call(
    _combine_body,
    grid=(N // BN,),
    in_specs=[
        pl.BlockSpec(memory_space=pltpu.MemorySpace.SMEM),
        pl.BlockSpec((BN, D), lambda i: (i, 0)),
        pl.BlockSpec((NC, BN, D), lambda i: (0, i, 0)),
    ],
    out_specs=pl.BlockSpec((BN, D), lambda i: (i, 0)),
    out_shape=jax.ShapeDtypeStruct((N, D), jnp.float32),
)


def kernel(h, pairs_0, pairs_1, degrees_0, degrees_1, scatter_idx,
           W_lin, b_lin, W_t, b_t, eps):
    p0 = pairs_0.astype(jnp.int32)
    p1 = pairs_1.astype(jnp.int32)
    si = scatter_idx.astype(jnp.int32)
    w_top = W_t[:D]
    w_deg = W_t[D:]
    h3, g = _matmuls(h, W_lin, b_lin.reshape(1, D), w_top, b_t.reshape(1, D))
    parts = _sc_gate_scatter(g, p0, p1, degrees_0.reshape(P * L),
                             degrees_1.reshape(P * L), si, -w_deg,
                             jnp.zeros((NPAD, D), jnp.float32))
    scale = (1.0 + eps).reshape(1, 1)
    return _combine(scale, h3, parts)
